# baseline (speedup 1.0000x reference)
import functools

import jax
import jax.numpy as jnp
from jax import lax
from jax.experimental import pallas as pl

K = 750.0
TOPK = 16
N = 100000
D = 64
M = 1024

BLK = 512
NB = (N + BLK - 1) // BLK
NPAD = NB * BLK


def _k1_body(xt_ref, x_ref, s_ref, bm_ref):
    i = pl.program_id(0)
    x = x_ref[...]
    ones = jnp.ones((1, D), jnp.float32)
    norms = lax.dot_general(ones, x * x, (((1,), (1,)), ((), ())),
                            preferred_element_type=jnp.float32)
    g = lax.dot_general(xt_ref[...], x, (((1,), (1,)), ((), ())),
                        preferred_element_type=jnp.float32)
    s = norms - 2.0 * g
    col = i * BLK + lax.broadcasted_iota(jnp.int32, (1, BLK), 1)
    s = jnp.where(col < N, s, jnp.float32(1e30))
    s_ref[...] = s
    bm_ref[0] = jnp.min(s, axis=1, keepdims=True)


def _stage1(xt, x):
    return pl.pallas_call(
        _k1_body,
        grid=(NB,),
        in_specs=[
            pl.BlockSpec((M, D), lambda i: (0, 0)),
            pl.BlockSpec((BLK, D), lambda i: (i, 0)),
        ],
        out_specs=[
            pl.BlockSpec((M, BLK), lambda i: (0, i)),
            pl.BlockSpec((1, M, 1), lambda i: (i, 0, 0)),
        ],
        out_shape=[
            jax.ShapeDtypeStruct((M, NPAD), jnp.float32),
            jax.ShapeDtypeStruct((NB, M, 1), jnp.float32),
        ],
    )(xt, x)


def kernel(X_tilde, X, w):
    xt = X_tilde.reshape(M, D)
    s, bm = _stage1(xt, X)
    idx = lax.top_k(-s[:, :N], TOPK)[1]
    nb = X[idx]
    l1 = jnp.sum(jnp.abs(nb - xt[:, None, :]), axis=2)
    act = w[idx].squeeze(2) - K * l1
    return jnp.max(act, axis=1, keepdims=True)


# speedup vs baseline: 1.8413x; 1.8413x over previous
import functools

import jax
import jax.numpy as jnp
from jax import lax
from jax.experimental import pallas as pl
from jax.experimental.pallas import tpu as pltpu
from jax.experimental.pallas import tpu_sc as plsc

K = 750.0
TOPK = 16
N = 100000
D = 64
M = 1024

BLK = 512
NB = (N + BLK - 1) // BLK
NPAD = NB * BLK
NBP = 208
NV_BM = NBP // 16
NV_CH = BLK // 16

NW = 32
QPW = M // NW


def _k1_body(xt_ref, x_ref, s_ref, bm_ref):
    i = pl.program_id(0)
    x = x_ref[...]
    ones = jnp.ones((1, D), jnp.float32)
    norms = lax.dot_general(ones, x * x, (((1,), (1,)), ((), ())),
                            preferred_element_type=jnp.float32)
    g = lax.dot_general(xt_ref[...], x, (((1,), (1,)), ((), ())),
                        preferred_element_type=jnp.float32)
    s = norms - 2.0 * g
    col = i * BLK + lax.broadcasted_iota(jnp.int32, (1, BLK), 1)
    s = jnp.where(col < N, s, jnp.float32(1e30))
    s_ref[:, 0, 0, :] = s
    bm_ref[0] = jnp.min(s, axis=1, keepdims=True)


def _stage1(xt, x):
    return pl.pallas_call(
        _k1_body,
        grid=(NB,),
        in_specs=[
            pl.BlockSpec((M, D), lambda i: (0, 0)),
            pl.BlockSpec((BLK, D), lambda i: (i, 0)),
        ],
        out_specs=[
            pl.BlockSpec((M, 1, 1, BLK), lambda i: (0, i, 0, 0)),
            pl.BlockSpec((1, M, 1), lambda i: (i, 0, 0)),
        ],
        out_shape=[
            jax.ShapeDtypeStruct((M, NB, 1, BLK), jnp.float32),
            jax.ShapeDtypeStruct((NB, M, 1), jnp.float32),
        ],
    )(xt, x)


def _merge16(Tv, Ti, v, vi, tmax):
    def merge(ops):
        Tv, Ti, v, vi = ops
        vs, vis = plsc.sort_key_val(v, vi)
        rvs = lax.rev(vs, (0,))
        rvi = lax.rev(vis, (0,))
        keep = Tv <= rvs
        nv = jnp.where(keep, Tv, rvs)
        ni = jnp.where(keep, Ti, rvi)
        Tv2, Ti2 = plsc.sort_key_val(nv, ni)
        return Tv2, Ti2, jnp.max(Tv2)

    def skip(ops):
        Tv, Ti, _, _ = ops
        return Tv, Ti, tmax

    return lax.cond(jnp.min(v) < tmax, merge, skip, (Tv, Ti, v, vi))


def _k2_body(s2d, bmp, xt, x2, w, out,
             bmrow, qv, cvals, xrows, wbuf, outb, sem):
    wid = lax.axis_index("s") * 2 + lax.axis_index("c")
    base = wid * QPW
    iota = lax.iota(jnp.int32, 16)
    inf = jnp.float32(3e38)
    pltpu.sync_copy(w, wbuf)

    def per_query(t, lane, res):
        q = base + t
        pltpu.sync_copy(bmp.at[q], bmrow)
        pltpu.sync_copy(xt.at[q], qv)

        def bm_step(j, carry):
            Tv, Ti, tmax = carry
            v = bmrow[pl.ds(j * 16, 16)]
            vi = j * 16 + iota
            return _merge16(Tv, Ti, v, vi, tmax)

        Tv0 = jnp.full((16,), inf, jnp.float32)
        Ti0 = jnp.zeros((16,), jnp.int32)
        Tv, Ti, _ = lax.fori_loop(0, NV_BM, bm_step, (Tv0, Ti0, inf))

        rows = q * NB + Ti
        pltpu.async_copy(s2d.at[rows], cvals, sem).wait()

        carry = (jnp.full((16,), inf, jnp.float32),
                 jnp.zeros((16,), jnp.int32), inf)
        for c in range(TOPK):
            cbase = Ti[c] * BLK

            def ch_step(j, carry, c=c, cbase=cbase):
                Nv, Ni, nmax = carry
                v = cvals[c, pl.ds(j * 16, 16)]
                vi = cbase + j * 16 + iota
                return _merge16(Nv, Ni, v, vi, nmax)

            carry = lax.fori_loop(0, NV_CH, ch_step, carry)
        _, Ni, _ = carry

        pltpu.async_copy(x2.at[lax.shift_right_logical(Ni, 1)], xrows,
                         sem).wait()
        wvreg = plsc.load_gather(wbuf, [Ni])

        qs = [qv[pl.ds(j * 16, 16)] for j in range(D // 16)]
        best = -inf
        for n in range(TOPK):
            off = (Ni[n] & 1) * D
            sv = jnp.zeros((16,), jnp.float32)
            for j in range(D // 16):
                sv = sv + jnp.abs(xrows[n, pl.ds(off + j * 16, 16)] - qs[j])
            act_n = wvreg[n] - jnp.float32(K) * jnp.sum(sv)
            best = jnp.maximum(best, act_n)
        return jnp.where(iota == lane, best, res)

    for h in range(2):
        res = lax.fori_loop(0, 16, lambda t, r, h=h: per_query(h * 16 + t, t, r),
                            jnp.zeros((16,), jnp.float32))
        outb[pl.ds(h * 16, 16)] = res
    pltpu.sync_copy(outb, out.at[pl.ds(base, QPW)])


_stage2 = functools.partial(
    pl.kernel,
    _k2_body,
    out_type=jax.ShapeDtypeStruct((M,), jnp.float32),
    mesh=plsc.VectorSubcoreMesh(core_axis_name="c", subcore_axis_name="s"),
    compiler_params=pltpu.CompilerParams(needs_layout_passes=False),
    scratch_types=[
        pltpu.VMEM((NBP,), jnp.float32),
        pltpu.VMEM((D,), jnp.float32),
        pltpu.VMEM((TOPK, BLK), jnp.float32),
        pltpu.VMEM((TOPK, 2 * D), jnp.float32),
        pltpu.VMEM((N,), jnp.float32),
        pltpu.VMEM((QPW,), jnp.float32),
        pltpu.SemaphoreType.DMA,
    ],
)()


def kernel(X_tilde, X, w):
    xt = X_tilde.reshape(M, D)
    s, bm3 = _stage1(xt, X)
    s2d = s.reshape(M * NB, BLK)
    bmp = jnp.pad(bm3.reshape(NB, M).T, ((0, 0), (0, NBP - NB)),
                  constant_values=jnp.float32(1e30))
    x2 = X.reshape(N // 2, 2 * D)
    dists = _stage2(s2d, bmp, xt, x2, w.reshape(N))
    return dists.reshape(M, 1)


# speedup vs baseline: 5.8203x; 3.1610x over previous
import functools

import jax
import jax.numpy as jnp
from jax import lax
from jax.experimental import pallas as pl
from jax.experimental.pallas import tpu as pltpu
from jax.experimental.pallas import tpu_sc as plsc

K = 750.0
TOPK = 16
N = 100000
D = 64
M = 1024

BLK = 2048
NB = (N + BLK - 1) // BLK
NPAD = NB * BLK
CH = 128
CPB = BLK // CH
NC = NB * CPB
NBP = 800
NV_BM = NBP // 16
NV_CH = CH // 16

NW = 32
QPW = M // NW


def _k1_body(xt_ref, x_ref, s_ref, bm_ref):
    i = pl.program_id(0)
    x = x_ref[...]
    ones = jnp.ones((1, D), jnp.float32)
    norms = lax.dot_general(ones, x * x, (((1,), (1,)), ((), ())),
                            preferred_element_type=jnp.float32)
    g = lax.dot_general(xt_ref[...], x, (((1,), (1,)), ((), ())),
                        preferred_element_type=jnp.float32)
    s = norms - 2.0 * g
    col = i * BLK + lax.broadcasted_iota(jnp.int32, (1, BLK), 1)
    s = jnp.where(col < N, s, jnp.float32(1e30))
    for c in range(CPB):
        sc = s[:, c * CH:(c + 1) * CH]
        s_ref[0, c] = sc
        bm_ref[0, c] = jnp.min(sc, axis=1, keepdims=True)


def _stage1(xt, x):
    return pl.pallas_call(
        _k1_body,
        grid=(NB,),
        in_specs=[
            pl.BlockSpec((M, D), lambda i: (0, 0)),
            pl.BlockSpec((BLK, D), lambda i: (i, 0)),
        ],
        out_specs=[
            pl.BlockSpec((1, CPB, M, CH), lambda i: (i, 0, 0, 0)),
            pl.BlockSpec((1, CPB, M, 1), lambda i: (i, 0, 0, 0)),
        ],
        out_shape=[
            jax.ShapeDtypeStruct((NB, CPB, M, CH), jnp.float32),
            jax.ShapeDtypeStruct((NB, CPB, M, 1), jnp.float32),
        ],
    )(xt, x)


def _merge16(Tv, Ti, v, vi, tmax):
    def merge(ops):
        Tv, Ti, v, vi = ops
        vs, vis = plsc.sort_key_val(v, vi)
        rvs = lax.rev(vs, (0,))
        rvi = lax.rev(vis, (0,))
        keep = Tv <= rvs
        nv = jnp.where(keep, Tv, rvs)
        ni = jnp.where(keep, Ti, rvi)
        Tv2, Ti2 = plsc.sort_key_val(nv, ni)
        return Tv2, Ti2, jnp.max(Tv2)

    def skip(ops):
        Tv, Ti, _, _ = ops
        return Tv, Ti, tmax

    return lax.cond(jnp.min(v) < tmax, merge, skip, (Tv, Ti, v, vi))


def _k2_body(s2d, bq, x2, w, out,
             bqrow, cvals, xrows, wbuf, outb, sem):
    wid = lax.axis_index("s") * 2 + lax.axis_index("c")
    base = wid * QPW
    iota = lax.iota(jnp.int32, 16)
    inf = jnp.float32(3e38)
    pltpu.sync_copy(w, wbuf)

    def per_query(t, lane, res):
        q = base + t
        pltpu.sync_copy(bq.at[q], bqrow)

        def bm_step(j, carry):
            Tv, Ti, tmax = carry
            v = bqrow[pl.ds(j * 16, 16)]
            vi = j * 16 + iota
            return _merge16(Tv, Ti, v, vi, tmax)

        Tv0 = jnp.full((16,), inf, jnp.float32)
        Ti0 = jnp.zeros((16,), jnp.int32)
        Tv, Ti, _ = lax.fori_loop(0, NV_BM, bm_step, (Tv0, Ti0, inf))

        rows = Ti * M + q
        pltpu.async_copy(s2d.at[rows], cvals, sem).wait()

        carry = (jnp.full((16,), inf, jnp.float32),
                 jnp.zeros((16,), jnp.int32), inf)
        for c in range(TOPK):
            cbase = Ti[c] * CH

            def ch_step(j, carry, c=c, cbase=cbase):
                Nv, Ni, nmax = carry
                v = cvals[c, pl.ds(j * 16, 16)]
                vi = cbase + j * 16 + iota
                return _merge16(Nv, Ni, v, vi, nmax)

            carry = lax.fori_loop(0, NV_CH, ch_step, carry)
        _, Ni, _ = carry

        pltpu.async_copy(x2.at[lax.shift_right_logical(Ni, 1)], xrows,
                         sem).wait()
        wvreg = plsc.load_gather(wbuf, [Ni])

        qs = [bqrow[pl.ds(NBP + j * 16, 16)] for j in range(D // 16)]
        best = -inf
        for n in range(TOPK):
            off = (Ni[n] & 1) * D
            sv = jnp.zeros((16,), jnp.float32)
            for j in range(D // 16):
                sv = sv + jnp.abs(xrows[n, pl.ds(off + j * 16, 16)] - qs[j])
            act_n = wvreg[n] - jnp.float32(K) * jnp.sum(sv)
            best = jnp.maximum(best, act_n)
        return jnp.where(iota == lane, best, res)

    for h in range(2):
        res = lax.fori_loop(0, 16, lambda t, r, h=h: per_query(h * 16 + t, t, r),
                            jnp.zeros((16,), jnp.float32))
        outb[pl.ds(h * 16, 16)] = res
    pltpu.sync_copy(outb, out.at[pl.ds(base, QPW)])


_stage2 = functools.partial(
    pl.kernel,
    _k2_body,
    out_type=jax.ShapeDtypeStruct((M,), jnp.float32),
    mesh=plsc.VectorSubcoreMesh(core_axis_name="c", subcore_axis_name="s"),
    compiler_params=pltpu.CompilerParams(needs_layout_passes=False),
    scratch_types=[
        pltpu.VMEM((NBP + D,), jnp.float32),
        pltpu.VMEM((TOPK, CH), jnp.float32),
        pltpu.VMEM((TOPK, 2 * D), jnp.float32),
        pltpu.VMEM((N,), jnp.float32),
        pltpu.VMEM((QPW,), jnp.float32),
        pltpu.SemaphoreType.DMA,
    ],
)()


def kernel(X_tilde, X, w):
    xt = X_tilde.reshape(M, D)
    s, bm4 = _stage1(xt, X)
    s2d = s.reshape(NC * M, CH)
    bmp = jnp.pad(bm4.reshape(NC, M).T, ((0, 0), (0, NBP - NC)),
                  constant_values=jnp.float32(1e30))
    x2 = X.reshape(N // 2, 2 * D)
    bq = jnp.concatenate([bmp, xt], axis=1)
    dists = _stage2(s2d, bq, x2, w.reshape(N))
    return dists.reshape(M, 1)
